# baseline (speedup 1.0000x reference)
import functools

import jax
import jax.numpy as jnp
from jax import lax
from jax.experimental import pallas as pl
from jax.experimental.pallas import tpu as pltpu
from jax.experimental.pallas import tpu_sc as plsc

VOCAB = 100000
EMB_DIM = 64
BATCH = 4096
SEQ = 50

_INFO = plsc.get_sparse_core_info()
NC = _INFO.num_cores
NS = _INFO.num_subcores
NW = NC * NS
CHUNK = 128

_MESH = plsc.VectorSubcoreMesh(core_axis_name="c", subcore_axis_name="s")


@functools.partial(
    pl.kernel,
    out_type=jax.ShapeDtypeStruct((SEQ, 8, NW, 8, CHUNK), jnp.float32),
    mesh=_MESH,
    scratch_types=[
        pltpu.VMEM((SEQ, CHUNK), jnp.int32),
        pltpu.VMEM((CHUNK, EMB_DIM), jnp.float32),
        pltpu.VMEM((CHUNK, EMB_DIM), jnp.float32),
        pltpu.VMEM((8, 8, CHUNK), jnp.float32),
        pltpu.VMEM((8, 8, CHUNK), jnp.float32),
        pltpu.SemaphoreType.DMA,
        pltpu.SemaphoreType.DMA,
        pltpu.SemaphoreType.DMA,
        pltpu.SemaphoreType.DMA,
    ],
    compiler_params=pltpu.CompilerParams(
        use_tc_tiling_on_sc=False, needs_layout_passes=False
    ),
)
def _gather_kernel(idx_hbm, table_hbm, out_hbm, idx_v, g0, g1, t0, t1,
                   sg0, sg1, sw0, sw1):
    wid = lax.axis_index("s") * NC + lax.axis_index("c")
    pltpu.sync_copy(idx_hbm.at[wid], idx_v)

    bufs_g = (g0, g1)
    bufs_t = (t0, t1)
    sems_g = (sg0, sg1)
    sems_w = (sw0, sw1)

    def fire_gather(s, p):
        return pltpu.async_copy(
            table_hbm.at[idx_v.at[s]], bufs_g[p], sems_g[p]
        )

    def transpose(p):
        src = bufs_g[p]
        dst = bufs_t[p]

        def body_g(g, _):
            rows = g * 16 + lax.iota(jnp.int32, 16)
            for e in range(EMB_DIM):
                cols = jnp.full((16,), e, jnp.int32)
                vals = plsc.load_gather(src, [rows, cols])
                dst[e // 8, e % 8, pl.ds(g * 16, 16)] = vals
            return 0

        lax.fori_loop(0, CHUNK // 16, body_g, 0)

    def fire_write(s, p):
        return pltpu.async_copy(
            bufs_t[p], out_hbm.at[s, pl.ds(0, 8), wid], sems_w[p]
        )

    fire_gather(0, 0)
    fire_gather(1, 1)

    def step(s2, _):
        for p in range(2):
            s = s2 * 2 + p
            pltpu.make_async_copy(
                table_hbm.at[idx_v.at[s]], bufs_g[p], sems_g[p]
            ).wait()

            @pl.when(s2 >= 1)
            def _():
                pltpu.make_async_copy(
                    bufs_t[p], out_hbm.at[s - 2, pl.ds(0, 8), wid], sems_w[p]
                ).wait()

            transpose(p)
            fire_write(s, p)

            @pl.when(s2 < SEQ // 2 - 1)
            def _():
                fire_gather(s + 2, p)
        return 0

    lax.fori_loop(0, SEQ // 2, step, 0)
    for p in range(2):
        pltpu.make_async_copy(
            bufs_t[p], out_hbm.at[SEQ - 2 + p, pl.ds(0, 8), wid], sems_w[p]
        ).wait()


def kernel(input_ids, table):
    idx5 = (
        input_ids.T.astype(jnp.int32)
        .reshape(SEQ, NW, CHUNK)
        .transpose(1, 0, 2)
    )
    o5 = _gather_kernel(idx5, table)
    return o5.transpose(2, 4, 0, 1, 3).reshape(BATCH, SEQ, EMB_DIM)


# speedup vs baseline: 1.4686x; 1.4686x over previous
import functools

import jax
import jax.numpy as jnp
from jax import lax
from jax.experimental import pallas as pl
from jax.experimental.pallas import tpu as pltpu
from jax.experimental.pallas import tpu_sc as plsc

VOCAB = 100000
EMB_DIM = 64
BATCH = 4096
SEQ = 50

_INFO = plsc.get_sparse_core_info()
NC = _INFO.num_cores
NS = _INFO.num_subcores
NW = NC * NS
CHUNK = 128

_MESH = plsc.VectorSubcoreMesh(core_axis_name="c", subcore_axis_name="s")


@functools.partial(
    pl.kernel,
    out_type=jax.ShapeDtypeStruct((SEQ, 8, NW, 8, CHUNK), jnp.float32),
    mesh=_MESH,
    scratch_types=[
        pltpu.VMEM((SEQ, CHUNK), jnp.int32),
        pltpu.VMEM((CHUNK, EMB_DIM), jnp.float32),
        pltpu.VMEM((CHUNK, EMB_DIM), jnp.float32),
        pltpu.VMEM((8, 8, CHUNK), jnp.float32),
        pltpu.VMEM((8, 8, CHUNK), jnp.float32),
        pltpu.SemaphoreType.DMA,
        pltpu.SemaphoreType.DMA,
        pltpu.SemaphoreType.DMA,
        pltpu.SemaphoreType.DMA,
    ],
    compiler_params=pltpu.CompilerParams(
        use_tc_tiling_on_sc=False, needs_layout_passes=False
    ),
)
def _gather_kernel(idx_hbm, table_hbm, out_hbm, idx_v, g0, g1, t0, t1,
                   sg0, sg1, sw0, sw1):
    wid = lax.axis_index("s") * NC + lax.axis_index("c")
    pltpu.sync_copy(idx_hbm.at[wid], idx_v)

    bufs_g = (g0, g1)
    bufs_t = (t0, t1)
    sems_g = (sg0, sg1)
    sems_w = (sw0, sw1)

    def fire_gather(s, p):
        return pltpu.async_copy(
            table_hbm.at[idx_v.at[s]], bufs_g[p], sems_g[p]
        )

    def transpose(p):
        src = bufs_g[p]
        dst = bufs_t[p]
        lanes = lax.iota(jnp.int32, 16)
        for c in range(EMB_DIM // 16):
            e_vec = c * 16 + lanes
            te_vec = e_vec // 8
            re_vec = e_vec % 8

            @plsc.parallel_loop(0, CHUNK, unroll=8)
            def _(rb):
                vals = src[rb, pl.ds(c * 16, 16)]
                rb_vec = jnp.full((16,), rb, jnp.int32)
                plsc.store_scatter(dst, [te_vec, re_vec, rb_vec], vals)

    def fire_write(s, p):
        return pltpu.async_copy(
            bufs_t[p], out_hbm.at[s, pl.ds(0, 8), wid], sems_w[p]
        )

    fire_gather(0, 0)
    fire_gather(1, 1)

    def step(s2, _):
        for p in range(2):
            s = s2 * 2 + p
            pltpu.make_async_copy(
                table_hbm.at[idx_v.at[s]], bufs_g[p], sems_g[p]
            ).wait()

            @pl.when(s2 >= 1)
            def _():
                pltpu.make_async_copy(
                    bufs_t[p], out_hbm.at[s - 2, pl.ds(0, 8), wid], sems_w[p]
                ).wait()

            transpose(p)
            fire_write(s, p)

            @pl.when(s2 < SEQ // 2 - 1)
            def _():
                fire_gather(s + 2, p)
        return 0

    lax.fori_loop(0, SEQ // 2, step, 0)
    for p in range(2):
        pltpu.make_async_copy(
            bufs_t[p], out_hbm.at[SEQ - 2 + p, pl.ds(0, 8), wid], sems_w[p]
        ).wait()


def kernel(input_ids, table):
    idx5 = (
        input_ids.T.astype(jnp.int32)
        .reshape(SEQ, NW, CHUNK)
        .transpose(1, 0, 2)
    )
    o5 = _gather_kernel(idx5, table)
    return o5.transpose(2, 4, 0, 1, 3).reshape(BATCH, SEQ, EMB_DIM)


# speedup vs baseline: 2.9288x; 1.9943x over previous
import functools

import jax
import jax.numpy as jnp
import numpy as np
from jax import lax
from jax.experimental import pallas as pl
from jax.experimental.pallas import tpu as pltpu
from jax.experimental.pallas import tpu_sc as plsc

VOCAB = 100000
EMB_DIM = 64
BATCH = 4096
SEQ = 50

_INFO = plsc.get_sparse_core_info()
NC = _INFO.num_cores
NS = _INFO.num_subcores
NW = NC * NS
CHUNK = 128

_MESH = plsc.VectorSubcoreMesh(core_axis_name="c", subcore_axis_name="s")


@functools.partial(
    pl.kernel,
    out_type=jax.ShapeDtypeStruct((SEQ, 8, NW, 8, CHUNK), jnp.float32),
    mesh=_MESH,
    scratch_types=[
        pltpu.VMEM((SEQ, CHUNK), jnp.int32),
        pltpu.VMEM((CHUNK, EMB_DIM), jnp.float32),
        pltpu.VMEM((CHUNK, EMB_DIM), jnp.float32),
        pltpu.VMEM((8, 8, CHUNK + 1), jnp.float32),
        pltpu.VMEM((8, 8, CHUNK + 1), jnp.float32),
        pltpu.SemaphoreType.DMA,
        pltpu.SemaphoreType.DMA,
        pltpu.SemaphoreType.DMA,
        pltpu.SemaphoreType.DMA,
    ],
    compiler_params=pltpu.CompilerParams(
        use_tc_tiling_on_sc=False, needs_layout_passes=False
    ),
)
def _gather_kernel(idx_hbm, table_hbm, out_hbm, idx_v, g0, g1, t0, t1,
                   sg0, sg1, sw0, sw1):
    wid = lax.axis_index("s") * NC + lax.axis_index("c")
    pltpu.sync_copy(idx_hbm.at[wid], idx_v)

    bufs_g = (g0, g1)
    bufs_t = (t0, t1)
    sems_g = (sg0, sg1)
    sems_w = (sw0, sw1)

    def fire_gather(s, p):
        return pltpu.async_copy(
            table_hbm.at[idx_v.at[s]], bufs_g[p], sems_g[p]
        )

    def transpose(p):
        src = bufs_g[p]
        dst = bufs_t[p]
        lanes = lax.iota(jnp.int32, 16)
        for c in range(EMB_DIM // 16):
            e_vec = c * 16 + lanes
            te_vec = e_vec // 8
            re_vec = e_vec % 8

            @plsc.parallel_loop(0, CHUNK, unroll=8)
            def _(rb):
                vals = src[rb, pl.ds(c * 16, 16)]
                rb_vec = jnp.full((16,), rb, jnp.int32)
                plsc.store_scatter(dst, [te_vec, re_vec, rb_vec], vals)

    def fire_write(s, p):
        for te in range(8):
            pltpu.async_copy(
                bufs_t[p].at[te, pl.ds(0, 8), pl.ds(0, CHUNK)],
                out_hbm.at[s, te, wid],
                sems_w[p],
            )

    fire_gather(0, 0)
    fire_gather(1, 1)

    def step(s2, _):
        for p in range(2):
            s = s2 * 2 + p
            pltpu.make_async_copy(
                table_hbm.at[idx_v.at[s]], bufs_g[p], sems_g[p]
            ).wait()

            @pl.when(s2 >= 1)
            def _():
                for te in range(8):
                    pltpu.make_async_copy(
                        bufs_t[p].at[te, pl.ds(0, 8), pl.ds(0, CHUNK)],
                        out_hbm.at[s - 2, te, wid],
                        sems_w[p],
                    ).wait()

            transpose(p)
            fire_write(s, p)

            @pl.when(s2 < SEQ // 2 - 1)
            def _():
                fire_gather(s + 2, p)
        return 0

    lax.fori_loop(0, SEQ // 2, step, 0)
    for p in range(2):
        for te in range(8):
            pltpu.make_async_copy(
                bufs_t[p].at[te, pl.ds(0, 8), pl.ds(0, CHUNK)],
                out_hbm.at[SEQ - 2 + p, te, wid],
                sems_w[p],
            ).wait()


def kernel(input_ids, table):
    idx5 = (
        input_ids.T.astype(jnp.int32)
        .reshape(SEQ, NW, CHUNK)
        .transpose(1, 0, 2)
    )
    o5 = _gather_kernel(idx5, table)
    return o5.transpose(2, 4, 0, 1, 3).reshape(BATCH, SEQ, EMB_DIM)
